# baseline (speedup 1.0000x reference)
import functools

import jax
import jax.numpy as jnp
from jax import lax
from jax.experimental import pallas as pl
from jax.experimental.pallas import tpu as pltpu
from jax.experimental.pallas import tpu_sc as plsc

EMB_DIM = 64
CHUNK = 128
NBUF = 4


@functools.lru_cache(maxsize=None)
def _build(B, NC, NS):
  NW = NC * NS
  b_per_w = B // NW
  n_chunks = b_per_w // CHUNK
  n_groups = n_chunks // NBUF
  mesh = plsc.VectorSubcoreMesh(core_axis_name="c", subcore_axis_name="s")

  @functools.partial(
      pl.kernel, mesh=mesh,
      compiler_params=pltpu.CompilerParams(use_tc_tiling_on_sc=False),
      out_type=jax.ShapeDtypeStruct((B, EMB_DIM), jnp.float32),
      scratch_types=(
          [pltpu.VMEM((n_chunks, CHUNK), jnp.int32)]
          + [pltpu.VMEM((CHUNK, EMB_DIM), jnp.float32) for _ in range(NBUF)]
          + [pltpu.SemaphoreType.DMA for _ in range(2 * NBUF)]
      ),
  )
  def gather_kernel(table_hbm, idx_hbm, out_hbm, idx_v, *rest):
    rows = rest[:NBUF]
    gsem = rest[NBUF:2 * NBUF]
    ssem = rest[2 * NBUF:]
    wid = lax.axis_index("s") * NC + lax.axis_index("c")
    base = wid * b_per_w

    pltpu.sync_copy(idx_hbm.at[wid], idx_v)

    def start_gather(g, b):
      pltpu.async_copy(table_hbm.at[idx_v.at[g]], rows[b], gsem[b])

    def wait_gather(g, b):
      pltpu.make_async_copy(table_hbm.at[idx_v.at[g]], rows[b], gsem[b]).wait()

    def start_store(g, b):
      pltpu.async_copy(rows[b], out_hbm.at[pl.ds(base + g * CHUNK, CHUNK)],
                       ssem[b])

    def wait_store(g, b):
      pltpu.make_async_copy(rows[b],
                            out_hbm.at[pl.ds(base + g * CHUNK, CHUNK)],
                            ssem[b]).wait()

    for b in range(NBUF):
      start_gather(b, b)

    def body(go, _):
      for b in range(NBUF):
        g = go * NBUF + b
        wait_gather(g, b)
        start_store(g, b)
        wait_store(g, b)
        start_gather(g + NBUF, b)
      return _

    lax.fori_loop(0, n_groups - 1, body, None)

    for b in range(NBUF):
      g = (n_groups - 1) * NBUF + b
      wait_gather(g, b)
      start_store(g, b)
    for b in range(NBUF):
      g = (n_groups - 1) * NBUF + b
      wait_store(g, b)

  return gather_kernel


def kernel(input_, weight):
  info = plsc.get_sparse_core_info()
  NC, NS = info.num_cores, info.num_subcores
  B = input_.size
  idx = input_.reshape(-1).astype(jnp.int32)
  idx3 = idx.reshape(NC * NS, -1, CHUNK)
  out = _build(B, NC, NS)(weight, idx3)
  return out.reshape(input_.shape + (EMB_DIM,))
